# Initial kernel scaffold; baseline (speedup 1.0000x reference)
import jax
import jax.numpy as jnp
from jax.experimental import pallas as pl


def kernel(x, edge_index, W1, b1, W2, b2, W3, b3):
    raise NotImplementedError("write your pallas kernel here")


# speedup vs baseline: 8.2940x; 8.2940x over previous
import functools

import jax
import jax.numpy as jnp
from jax import lax
from jax.experimental import pallas as pl
from jax.experimental.pallas import tpu as pltpu
from jax.experimental.pallas import tpu_sc as plsc

N = 10000
D = 128
HD = 64
E = 320000
TEMP = 10.0
WEIGHT = 1.0

NC = 2
NS = 16
CH = 128
NCH = 160
HCH = NCH // NC
EPT = CH * NCH
EP = EPT * NS
NACC = 10240
RPT = NACC // NS
SDUMP = N

_mesh = plsc.VectorSubcoreMesh(core_axis_name="c", subcore_axis_name="s")


def _fill(ref, rows, cols, value):
    def body(i, _):
        r = i // (cols // 16)
        c = (i % (cols // 16)) * 16
        ref[r, pl.ds(c, 16)] = jnp.full((16,), value, jnp.float32)
        return 0
    lax.fori_loop(0, rows * (cols // 16), body, 0)


def _sc_pass_body(has_vec, has_scal, gather_ones, refs):
    it = iter(refs)
    tab_lo = next(it) if has_vec else None
    tab_hi = next(it) if has_vec else None
    tab16 = next(it) if (has_scal and not gather_ones) else None
    gidx_hbm = next(it)
    sidx_hbm = next(it)
    out_lo = next(it) if has_vec else None
    out_hi = next(it) if has_vec else None
    out16 = next(it) if has_scal else None
    gidx_v = next(it)
    sidx_v = next(it)
    if has_vec:
        bufA = next(it); bufB = next(it)
        acc = next(it)
        zb = next(it)
        gsA = next(it); gsB = next(it)
        ssA = next(it); ssB = next(it)
    if has_scal:
        sbufA = next(it); sbufB = next(it)
        acc16 = next(it)
        zb16 = next(it)
        g16A = next(it); g16B = next(it)
        s16A = next(it); s16B = next(it)

    cid = lax.axis_index("c")
    sid = lax.axis_index("s")
    row0 = sid * RPT

    pltpu.sync_copy(gidx_hbm.at[sid], gidx_v)
    pltpu.sync_copy(sidx_hbm.at[sid], sidx_v)

    if has_vec:
        _fill(zb, CH, HD, 0.0)
        for k in range(RPT // CH):
            pltpu.sync_copy(zb, acc.at[pl.ds(row0 + k * CH, CH)])
    if has_scal:
        _fill(zb16, CH, 16, 0.0)
        for k in range(RPT // CH):
            pltpu.sync_copy(zb16, acc16.at[pl.ds(row0 + k * CH, CH)])
    if gather_ones:
        _fill(sbufA, CH, 16, 1.0)
    plsc.subcore_barrier()

    if has_vec:
        def g_start(c, buf, gs):
            @pl.when(cid == 0)
            def _():
                pltpu.async_copy(tab_lo.at[gidx_v.at[c]], buf, gs)
            @pl.when(cid == 1)
            def _():
                pltpu.async_copy(tab_hi.at[gidx_v.at[c]], buf, gs)

        def g_wait(buf, gs):
            pltpu.make_async_copy(tab_lo.at[gidx_v.at[0]], buf, gs).wait()

        g_start(0, bufA, gsA)
        g_start(1, bufB, gsB)

        def loop(t, _):
            c0 = 2 * t
            for (cc, buf, gs, ss) in ((c0, bufA, gsA, ssA), (c0 + 1, bufB, gsB, ssB)):
                g_wait(buf, gs)
                pltpu.async_copy(buf, acc.at[sidx_v.at[cc]], ss, add=True)
                pltpu.make_async_copy(buf, acc.at[sidx_v.at[0]], ss).wait()
                @pl.when(cc + 2 < NCH)
                def _():
                    g_start(cc + 2, buf, gs)
            return 0
        lax.fori_loop(0, NCH // 2, loop, 0)

    if has_scal:
        base = cid * HCH
        if gather_ones:
            def sloop(t, _):
                c0 = base + 2 * t
                pltpu.async_copy(sbufA, acc16.at[sidx_v.at[c0]], s16A, add=True)
                pltpu.async_copy(sbufA, acc16.at[sidx_v.at[c0 + 1]], s16B, add=True)
                pltpu.make_async_copy(sbufA, acc16.at[sidx_v.at[0]], s16A).wait()
                pltpu.make_async_copy(sbufA, acc16.at[sidx_v.at[0]], s16B).wait()
                return 0
            lax.fori_loop(0, HCH // 2, sloop, 0)
        else:
            pltpu.async_copy(tab16.at[gidx_v.at[base]], sbufA, g16A)
            pltpu.async_copy(tab16.at[gidx_v.at[base + 1]], sbufB, g16B)

            def sloop(t, _):
                c0 = base + 2 * t
                for (cc, sbuf, gs, ss) in ((c0, sbufA, g16A, s16A),
                                           (c0 + 1, sbufB, g16B, s16B)):
                    pltpu.make_async_copy(tab16.at[gidx_v.at[0]], sbuf, gs).wait()
                    pltpu.async_copy(sbuf, acc16.at[sidx_v.at[cc]], ss, add=True)
                    pltpu.make_async_copy(sbuf, acc16.at[sidx_v.at[0]], ss).wait()
                    @pl.when(cc + 2 < base + HCH)
                    def _():
                        pltpu.async_copy(tab16.at[gidx_v.at[cc + 2]], sbuf, gs)
                return 0
            lax.fori_loop(0, HCH // 2, sloop, 0)

    plsc.subcore_barrier()

    if has_vec:
        @pl.when(cid == 0)
        def _():
            pltpu.sync_copy(acc.at[pl.ds(row0, RPT)], out_lo.at[pl.ds(row0, RPT)])
        @pl.when(cid == 1)
        def _():
            pltpu.sync_copy(acc.at[pl.ds(row0, RPT)], out_hi.at[pl.ds(row0, RPT)])
    if has_scal:
        pltpu.sync_copy(acc16.at[pl.ds(row0, RPT)], out16.at[cid, pl.ds(row0, RPT)])


def _make_sc_pass(has_vec, has_scal, gather_ones=False):
    out_type = []
    if has_vec:
        out_type.append(jax.ShapeDtypeStruct((NACC, HD), jnp.float32))
        out_type.append(jax.ShapeDtypeStruct((NACC, HD), jnp.float32))
    if has_scal:
        out_type.append(jax.ShapeDtypeStruct((NC, NACC, 16), jnp.float32))
    scratch = [
        pltpu.VMEM((NCH, CH), jnp.int32),
        pltpu.VMEM((NCH, CH), jnp.int32),
    ]
    if has_vec:
        scratch += [
            pltpu.VMEM((CH, HD), jnp.float32),
            pltpu.VMEM((CH, HD), jnp.float32),
            pltpu.VMEM_SHARED((NACC, HD), jnp.float32),
            pltpu.VMEM((CH, HD), jnp.float32),
            pltpu.SemaphoreType.DMA, pltpu.SemaphoreType.DMA,
            pltpu.SemaphoreType.DMA, pltpu.SemaphoreType.DMA,
        ]
    if has_scal:
        scratch += [
            pltpu.VMEM((CH, 16), jnp.float32),
            pltpu.VMEM((CH, 16), jnp.float32),
            pltpu.VMEM_SHARED((NACC, 16), jnp.float32),
            pltpu.VMEM((CH, 16), jnp.float32),
            pltpu.SemaphoreType.DMA, pltpu.SemaphoreType.DMA,
            pltpu.SemaphoreType.DMA, pltpu.SemaphoreType.DMA,
        ]
    body = functools.partial(_sc_pass_body, has_vec, has_scal, gather_ones)
    return pl.kernel(
        lambda *refs: body(refs),
        out_type=out_type if len(out_type) > 1 else out_type[0],
        mesh=_mesh,
        scratch_types=scratch,
        compiler_params=pltpu.CompilerParams(use_tc_tiling_on_sc=False),
    )


_sc_deg = _make_sc_pass(False, True, gather_ones=True)
_sc_vec = _make_sc_pass(True, False)
_sc_vecscal = _make_sc_pass(True, True)


RB = 2000
GRID = N // RB

_b2 = lambda i: (i, 0)
_bfull = lambda i: (0, 0)
_b3 = lambda i: (0, i, 0)

_f32 = jnp.float32


def _spec_rows(cols):
    return pl.BlockSpec((RB, cols), _b2)


def _stageA_body(x_ref, w_ref, degp_ref, dinv_ref, degin_ref, lo_ref, hi_ref):
    deg = degp_ref[0, :, 0] + degp_ref[1, :, 0] + 1.0
    dinv = lax.rsqrt(deg)
    dinv_ref[...] = jnp.broadcast_to(dinv[:, None], (RB, 16))
    degin_ref[...] = jnp.broadcast_to((deg - 1.0)[:, None], (RB, 16))
    h2d = dinv[:, None] * jnp.dot(x_ref[...], w_ref[...],
                                  preferred_element_type=_f32)
    lo_ref[...] = h2d[:, :HD]
    hi_ref[...] = h2d[:, HD:]


_stageA = pl.pallas_call(
    _stageA_body,
    grid=(GRID,),
    in_specs=[
        _spec_rows(D),
        pl.BlockSpec((D, D), _bfull),
        pl.BlockSpec((NC, RB, 16), _b3),
    ],
    out_specs=[_spec_rows(16), _spec_rows(16), _spec_rows(HD), _spec_rows(HD)],
    out_shape=[
        jax.ShapeDtypeStruct((N, 16), _f32),
        jax.ShapeDtypeStruct((N, 16), _f32),
        jax.ShapeDtypeStruct((N, HD), _f32),
        jax.ShapeDtypeStruct((N, HD), _f32),
    ],
)


def _stageB_body(mlo_ref, mhi_ref, hlo_ref, hhi_ref, dinv_ref, b_ref,
                 olo_ref, ohi_ref, sq_ref):
    dinv = dinv_ref[:, 0][:, None]
    lo = dinv * (mlo_ref[...] + hlo_ref[...]) + b_ref[...][None, :HD]
    hi = dinv * (mhi_ref[...] + hhi_ref[...]) + b_ref[...][None, HD:]
    olo_ref[...] = lo
    ohi_ref[...] = hi
    sq = jnp.sum(lo * lo, axis=1, keepdims=True) + jnp.sum(hi * hi, axis=1, keepdims=True)
    sq_ref[...] = jnp.broadcast_to(sq, (RB, 16))


_stageB = pl.pallas_call(
    _stageB_body,
    grid=(GRID,),
    in_specs=[
        _spec_rows(HD), _spec_rows(HD), _spec_rows(HD), _spec_rows(HD),
        _spec_rows(16),
        pl.BlockSpec((D,), lambda i: (0,)),
    ],
    out_specs=[_spec_rows(HD), _spec_rows(HD), _spec_rows(16)],
    out_shape=[
        jax.ShapeDtypeStruct((N, HD), _f32),
        jax.ShapeDtypeStruct((N, HD), _f32),
        jax.ShapeDtypeStruct((N, 16), _f32),
    ],
)


def _stageE_body(mlo_ref, mhi_ref, hlo_ref, hhi_ref, dinv_ref, b_ref, out_ref):
    dinv = dinv_ref[:, 0][:, None]
    lo = dinv * (mlo_ref[...] + hlo_ref[...]) + b_ref[...][None, :HD]
    hi = dinv * (mhi_ref[...] + hhi_ref[...]) + b_ref[...][None, HD:]
    out_ref[...] = jnp.concatenate([lo, hi], axis=1)


_stageE = pl.pallas_call(
    _stageE_body,
    grid=(GRID,),
    in_specs=[
        _spec_rows(HD), _spec_rows(HD), _spec_rows(HD), _spec_rows(HD),
        _spec_rows(16),
        pl.BlockSpec((D,), lambda i: (0,)),
    ],
    out_specs=_spec_rows(D),
    out_shape=jax.ShapeDtypeStruct((N, D), _f32),
)


def _stageC1_body(mlo_ref, mhi_ref, smp_ref, hlo_ref, hhi_ref, sq_ref,
                  degin_ref, en_ref):
    sm = smp_ref[0, :, 0] + smp_ref[1, :, 0]
    hm = (jnp.sum(hlo_ref[...] * mlo_ref[...], axis=1)
          + jnp.sum(hhi_ref[...] * mhi_ref[...], axis=1))
    En = 0.5 * sm + 0.5 * degin_ref[:, 0] * sq_ref[:, 0] - hm
    en_ref[...] = jnp.broadcast_to(En[:, None], (RB, 16))


_stageC1 = pl.pallas_call(
    _stageC1_body,
    grid=(GRID,),
    in_specs=[
        _spec_rows(HD), _spec_rows(HD),
        pl.BlockSpec((NC, RB, 16), _b3),
        _spec_rows(HD), _spec_rows(HD),
        _spec_rows(16), _spec_rows(16),
    ],
    out_specs=_spec_rows(16),
    out_shape=jax.ShapeDtypeStruct((N, 16), _f32),
)


def _stageC2_body(en_ref, v16_ref):
    En = en_ref[:, 0]
    Z = jnp.sum(En) + 1e-12
    Ens = En / Z
    L = -Ens / TEMP
    eL = jnp.exp(L - jnp.max(L))
    P = eL / jnp.sum(eL)
    g = -(jnp.log(P + 1e-12) + P / (P + 1e-12))
    dSdL = P * (g - jnp.sum(P * g))
    u = -dSdL / TEMP
    v = (u - jnp.sum(u * Ens)) / Z
    v16_ref[...] = jnp.broadcast_to(v[:, None], (N, 16))


_stageC2 = pl.pallas_call(
    _stageC2_body,
    grid=(1,),
    in_specs=[pl.BlockSpec((N, 16), _bfull)],
    out_specs=pl.BlockSpec((N, 16), _bfull),
    out_shape=jax.ShapeDtypeStruct((N, 16), _f32),
)


def _stageC3_body(hlo_ref, hhi_ref, v16_ref, qlo_ref, qhi_ref):
    v = v16_ref[:, 0][:, None]
    qlo_ref[...] = v * hlo_ref[...]
    qhi_ref[...] = v * hhi_ref[...]


_stageC3 = pl.pallas_call(
    _stageC3_body,
    grid=(GRID,),
    in_specs=[_spec_rows(HD), _spec_rows(HD), _spec_rows(16)],
    out_specs=[_spec_rows(HD), _spec_rows(HD)],
    out_shape=[
        jax.ShapeDtypeStruct((N, HD), _f32),
        jax.ShapeDtypeStruct((N, HD), _f32),
    ],
)


def _stageC(m_lo, m_hi, smP, blo, bhi, sq16, degin):
    en16 = _stageC1(m_lo, m_hi, smP, blo, bhi, sq16, degin)
    v16 = _stageC2(en16)
    qlo, qhi = _stageC3(blo, bhi, v16)
    return qlo, qhi, v16


def _stageD_body(wlo_ref, whi_ref, ap_ref, v16_ref, degin_ref,
                 hlo_ref, hhi_ref, mlo_ref, mhi_ref, dinv_ref, w2_ref,
                 olo_ref, ohi_ref):
    a = ap_ref[0, :, 0] + ap_ref[1, :, 0]
    v = v16_ref[:, 0]
    s = (a + v * degin_ref[:, 0])[:, None]
    vv = v[:, None]
    hlo = hlo_ref[...]
    hhi = hhi_ref[...]
    glo = s * hlo - wlo_ref[...] - vv * mlo_ref[...]
    ghi = s * hhi - whi_ref[...] - vv * mhi_ref[...]
    hn = jnp.concatenate([
        jnp.maximum(hlo + (WEIGHT * TEMP) * glo, 0.0),
        jnp.maximum(hhi + (WEIGHT * TEMP) * ghi, 0.0),
    ], axis=1)
    out = dinv_ref[:, 0][:, None] * jnp.dot(hn, w2_ref[...],
                                            preferred_element_type=_f32)
    olo_ref[...] = out[:, :HD]
    ohi_ref[...] = out[:, HD:]


_stageD = pl.pallas_call(
    _stageD_body,
    grid=(GRID,),
    in_specs=[
        _spec_rows(HD), _spec_rows(HD),
        pl.BlockSpec((NC, RB, 16), _b3),
        _spec_rows(16), _spec_rows(16),
        _spec_rows(HD), _spec_rows(HD), _spec_rows(HD), _spec_rows(HD),
        _spec_rows(16),
        pl.BlockSpec((D, D), _bfull),
    ],
    out_specs=[_spec_rows(HD), _spec_rows(HD)],
    out_shape=[
        jax.ShapeDtypeStruct((N, HD), _f32),
        jax.ShapeDtypeStruct((N, HD), _f32),
    ],
)


def kernel(x, edge_index, W1, b1, W2, b2, W3, b3):
    src = edge_index[0].astype(jnp.int32)
    dst = edge_index[1].astype(jnp.int32)
    pad = EP - E
    src0 = jnp.pad(src, (0, pad)).reshape(NS, NCH, CH)
    srcN = jnp.pad(src, (0, pad), constant_values=SDUMP).reshape(NS, NCH, CH)
    dst0 = jnp.pad(dst, (0, pad)).reshape(NS, NCH, CH)
    dstN = jnp.pad(dst, (0, pad), constant_values=SDUMP).reshape(NS, NCH, CH)

    degP = _sc_deg(dstN, dstN)
    dinv, degin, h2d_lo, h2d_hi = _stageA(x, W1, degP)

    def layer(hlo, hhi, b, Wnext):
        mp_lo, mp_hi = _sc_vec(hlo, hhi, src0, dstN)
        blo, bhi, sq16 = _stageB(mp_lo, mp_hi, hlo, hhi, dinv, b)
        m_lo, m_hi, smP = _sc_vecscal(blo, bhi, sq16, src0, dstN)
        qlo, qhi, v16 = _stageC(m_lo, m_hi, smP, blo, bhi, sq16, degin)
        w_lo, w_hi, aP = _sc_vecscal(qlo, qhi, v16, dst0, srcN)
        return _stageD(w_lo, w_hi, aP, v16, degin, blo, bhi, m_lo, m_hi,
                       dinv, Wnext)

    h2d_lo, h2d_hi = layer(h2d_lo, h2d_hi, b1, W2)
    h3d_lo, h3d_hi = layer(h2d_lo, h2d_hi, b2, W3)
    mp_lo, mp_hi = _sc_vec(h3d_lo, h3d_hi, src0, dstN)
    return _stageE(mp_lo, mp_hi, h3d_lo, h3d_hi, dinv, b3)
